# Initial kernel scaffold; baseline (speedup 1.0000x reference)
import jax
import jax.numpy as jnp
from jax.experimental import pallas as pl


def kernel(feat_word, feat_topic, ei_ww, ei_wt, ei_wd, ei_td, ei_tt, w_ww, w_wt, w_wd, w_td, w_tt, W_ww, b_ww, W_wt, b_wt, W_wd, b_wd, W_td, b_td, W_tt, b_tt):
    raise NotImplementedError("write your pallas kernel here")


# speedup vs baseline: 1.6102x; 1.6102x over previous
import functools

import jax
import jax.numpy as jnp
from jax import lax
from jax.experimental import pallas as pl
from jax.experimental.pallas import tpu as pltpu
from jax.experimental.pallas import tpu_sc as plsc

_NW, _NT, _ND = 50000, 5000, 10000
_D = 128
_CC = 32
_NCH = _D // _CC
_B = 128
_NTILES = 16

_NWP = 50176
_NTP = 5120
_NDP = 10240


def _pad_edges(ei, w, n_dst, mult):
    ne = ei.shape[1]
    ne_pad = ((ne + mult - 1) // mult) * mult
    pad = ne_pad - ne
    src = jnp.concatenate([ei[0], jnp.zeros((pad,), jnp.int32)])
    dst = jnp.concatenate([ei[1], jnp.full((pad,), n_dst, jnp.int32)])
    wp = jnp.concatenate([w, jnp.zeros((pad,), jnp.float32)])
    return src, dst, wp


def _sc_aggregate(fwc, ftc, eww, ewt, ewd, etd, ett):
    mesh = plsc.VectorSubcoreMesh(core_axis_name="c", subcore_axis_name="s")
    f32 = jnp.float32

    def etype_out(n_pad):
        return [jax.ShapeDtypeStruct((n_pad, _CC), f32) for _ in range(5)]

    out_type = (etype_out(_NWP) + etype_out(_NTP) + etype_out(_NTP)
                + etype_out(_NDP) + etype_out(_NDP))
    scratch_types = [
        pltpu.VMEM((_B, _CC), f32),
        pltpu.VMEM((_B,), jnp.int32),
        pltpu.VMEM((_B,), jnp.int32),
        pltpu.VMEM((_B,), f32),
        pltpu.VMEM_SHARED((_NWP, _CC), f32),
    ]

    @functools.partial(pl.kernel, out_type=out_type, mesh=mesh,
                       scratch_types=scratch_types,
                       compiler_params=pltpu.CompilerParams(
                           use_tc_tiling_on_sc=False))
    def agg(fw0, fw1, fw2, fw3, ft0, ft1, ft2, ft3,
            s_ww, d_ww, v_ww, s_wt, d_wt, v_wt, s_wd, d_wd, v_wd,
            s_td, d_td, v_td, s_tt, d_tt, v_tt,
            Sw0, Sw1, Sw2, Sw3, Pw,
            Swt0, Swt1, Swt2, Swt3, Pwt,
            Stt0, Stt1, Stt2, Stt3, Ptt,
            Swd0, Swd1, Swd2, Swd3, Pwd,
            Std0, Std1, Std2, Std3, Ptd,
            rows, sblk, dblk, wblk, acc):
        core = lax.axis_index("c")
        tid = lax.axis_index("s")

        lane = lax.broadcasted_iota(jnp.int32, (16,), 0)
        e0f = jnp.where(lane == 0, 1.0, 0.0)
        e1f = jnp.where(lane == 1, 1.0, 0.0)
        zeros16 = jnp.zeros((16,), f32)

        def zero_rows():
            @pl.loop(0, _B)
            def _(j):
                rows[j, pl.ds(0, 16)] = zeros16
                rows[j, pl.ds(16, 16)] = zeros16

        def zero_acc(n_pad):
            zr = n_pad // _NTILES
            zf, zrem = divmod(zr, _B)
            for i in range(zf):
                pltpu.sync_copy(rows, acc.at[pl.ds(tid * zr + i * _B, _B)])
            if zrem:
                pltpu.sync_copy(rows.at[pl.ds(0, zrem)],
                                acc.at[pl.ds(tid * zr + zf * _B, zrem)])

        def copy_out(n_pad, out_ref):
            zr = n_pad // _NTILES
            zf, zrem = divmod(zr, _B)
            for i in range(zf):
                o = tid * zr + i * _B
                pltpu.sync_copy(acc.at[pl.ds(o, _B)], out_ref.at[pl.ds(o, _B)])
            if zrem:
                o = tid * zr + zf * _B
                pltpu.sync_copy(acc.at[pl.ds(o, zrem)],
                                out_ref.at[pl.ds(o, zrem)])

        def one_pass(F, sref, dref, wref, out_ref, n_pad, pairs_mode):
            slice_len = sref.shape[0] // _NTILES
            nblk = slice_len // _B
            zero_rows()
            zero_acc(n_pad)
            plsc.subcore_barrier()

            @pl.loop(0, nblk)
            def _(b):
                off = tid * slice_len + b * _B
                pltpu.sync_copy(dref.at[pl.ds(off, _B)], dblk)
                pltpu.sync_copy(wref.at[pl.ds(off, _B)], wblk)
                if pairs_mode:
                    @pl.loop(0, _B // 16)
                    def _(g):
                        wv = wblk[pl.ds(g * 16, 16)]
                        for j in range(16):
                            rows[g * 16 + j, pl.ds(0, 16)] = e0f * wv[j] + e1f
                else:
                    pltpu.sync_copy(sref.at[pl.ds(off, _B)], sblk)
                    pltpu.sync_copy(F.at[sblk], rows)

                    @pl.loop(0, _B // 16)
                    def _(g):
                        wv = wblk[pl.ds(g * 16, 16)]
                        for j in range(16):
                            ws = wv[j]
                            jr = g * 16 + j
                            rows[jr, pl.ds(0, 16)] = rows[jr, pl.ds(0, 16)] * ws
                            rows[jr, pl.ds(16, 16)] = (
                                rows[jr, pl.ds(16, 16)] * ws)

                pltpu.sync_copy(rows, acc.at[dblk], add=True)

            plsc.subcore_barrier()
            copy_out(n_pad, out_ref)
            plsc.subcore_barrier()

        def etype(Fc, sref, dref, wref, outs, n_pad):
            for c in range(_NCH):
                one_pass(Fc[c], sref, dref, wref, outs[c], n_pad, False)
            one_pass(Fc[0], sref, dref, wref, outs[4], n_pad, True)

        fwc_ = [fw0, fw1, fw2, fw3]
        ftc_ = [ft0, ft1, ft2, ft3]

        @pl.when(core == 0)
        def _():
            etype(fwc_, s_ww, d_ww, v_ww, [Sw0, Sw1, Sw2, Sw3, Pw], _NWP)

        @pl.when(core == 1)
        def _():
            etype(fwc_, s_wt, d_wt, v_wt, [Swt0, Swt1, Swt2, Swt3, Pwt], _NTP)
            etype(ftc_, s_tt, d_tt, v_tt, [Stt0, Stt1, Stt2, Stt3, Ptt], _NTP)
            etype(fwc_, s_wd, d_wd, v_wd, [Swd0, Swd1, Swd2, Swd3, Pwd], _NDP)
            etype(ftc_, s_td, d_td, v_td, [Std0, Std1, Std2, Std3, Ptd], _NDP)

    outs = agg(*fwc, *ftc, *eww, *ewt, *ewd, *etd, *ett)
    return [outs[i * 5:i * 5 + 5] for i in range(5)]


_TBLK = 512


def _etype_specs():
    return ([pl.BlockSpec((_TBLK, _CC), lambda i: (i, 0)) for _ in range(5)]
            + [pl.BlockSpec((_D, _D), lambda i: (0, 0)),
               pl.BlockSpec((1, _D), lambda i: (0, 0))])


def _mean_part(srefs, p_ref, wt_ref, b_ref):
    acc = jnp.dot(srefs[0][...], wt_ref[pl.ds(0, _CC), :],
                  preferred_element_type=jnp.float32)
    for c in range(1, _NCH):
        acc += jnp.dot(srefs[c][...], wt_ref[pl.ds(c * _CC, _CC), :],
                       preferred_element_type=jnp.float32)
    sw = p_ref[:, 0:1]
    cnt = p_ref[:, 1:2]
    return (acc + sw * b_ref[...]) / jnp.maximum(cnt, 1.0)


def _tc_body1(s0, s1, s2, s3, p, wt, b, o_ref):
    o_ref[...] = _mean_part([s0, s1, s2, s3], p, wt, b)


def _tc_body2(a0, a1, a2, a3, ap, awt, ab, b0, b1, b2, b3, bp, bwt, bb, o_ref):
    o_ref[...] = (_mean_part([a0, a1, a2, a3], ap, awt, ab)
                  + _mean_part([b0, b1, b2, b3], bp, bwt, bb))


def _tc_combine1(agg5, Wt, b):
    n = agg5[0].shape[0]
    return pl.pallas_call(
        _tc_body1,
        grid=(n // _TBLK,),
        in_specs=_etype_specs(),
        out_specs=pl.BlockSpec((_TBLK, _D), lambda i: (i, 0)),
        out_shape=jax.ShapeDtypeStruct((n, _D), jnp.float32),
    )(*agg5, Wt, b)


def _tc_combine2(agg5a, Wta, ba, agg5b, Wtb, bb):
    n = agg5a[0].shape[0]
    return pl.pallas_call(
        _tc_body2,
        grid=(n // _TBLK,),
        in_specs=_etype_specs() + _etype_specs(),
        out_specs=pl.BlockSpec((_TBLK, _D), lambda i: (i, 0)),
        out_shape=jax.ShapeDtypeStruct((n, _D), jnp.float32),
    )(*agg5a, Wta, ba, *agg5b, Wtb, bb)


def kernel(feat_word, feat_topic, ei_ww, ei_wt, ei_wd, ei_td, ei_tt,
           w_ww, w_wt, w_wd, w_td, w_tt,
           W_ww, b_ww, W_wt, b_wt, W_wd, b_wd, W_td, b_td, W_tt, b_tt):
    mult = _NTILES * _B
    eww = _pad_edges(ei_ww, w_ww, _NW, mult)
    ewt = _pad_edges(ei_wt, w_wt, _NT, mult)
    ewd = _pad_edges(ei_wd, w_wd, _ND, mult)
    etd = _pad_edges(ei_td, w_td, _ND, mult)
    ett = _pad_edges(ei_tt, w_tt, _NT, mult)
    fwc = [feat_word[:, c * _CC:(c + 1) * _CC] for c in range(_NCH)]
    ftc = [feat_topic[:, c * _CC:(c + 1) * _CC] for c in range(_NCH)]

    agg_ww, agg_wt, agg_tt, agg_wd, agg_td = \
        _sc_aggregate(fwc, ftc, eww, ewt, ewd, etd, ett)

    h_word = _tc_combine1(agg_ww, W_ww.T, b_ww.reshape(1, _D))
    h_topic = _tc_combine2(agg_wt, W_wt.T, b_wt.reshape(1, _D),
                           agg_tt, W_tt.T, b_tt.reshape(1, _D))
    h_doc = _tc_combine2(agg_wd, W_wd.T, b_wd.reshape(1, _D),
                         agg_td, W_td.T, b_td.reshape(1, _D))
    return (h_word[:_NW], h_topic[:_NT], h_doc[:_ND])
